# baseline (speedup 1.0000x reference)
import functools

import jax
import jax.numpy as jnp
from jax.experimental import pallas as pl
from jax.experimental.pallas import tpu as pltpu
from jax.experimental.pallas import tpu_sc as plsc

S = 16

_NC, _NS, _NL = 2, 16, 16
_NW = _NC * _NS


def _sc_histogram(F):
    per_w = F // _NW
    mesh = plsc.VectorSubcoreMesh(core_axis_name="c", subcore_axis_name="s")

    @functools.partial(
        pl.kernel, mesh=mesh,
        out_type=jax.ShapeDtypeStruct((_NC, S), jnp.float32),
        scratch_types=[
            pltpu.VMEM((per_w,), jnp.int32),
            pltpu.VMEM((per_w,), jnp.float32),
            pltpu.VMEM((S,), jnp.float32),
            pltpu.VMEM_SHARED((S,), jnp.float32),
        ],
    )
    def hist(idx_hbm, out_hbm, idx_v, ones_v, zero_v, hist_sh):
        cid = jax.lax.axis_index("c")
        sid = jax.lax.axis_index("s")
        wid = sid * _NC + cid
        pltpu.sync_copy(idx_hbm.at[pl.ds(wid * per_w, per_w)], idx_v)

        @pl.when(sid == 0)
        def _():
            zero_v[...] = jnp.zeros((S,), jnp.float32)
            pltpu.sync_copy(zero_v, hist_sh)

        plsc.subcore_barrier()
        ones_v[...] = jnp.ones((per_w,), jnp.float32)
        pltpu.sync_copy(ones_v, hist_sh.at[idx_v], add=True)
        plsc.subcore_barrier()

        @pl.when(sid == 0)
        def _():
            pltpu.sync_copy(hist_sh, out_hbm.at[cid])

    return hist


def _fused(bi_ref, cp_ref, x_hbm, w_ref, out_ref, gc_ref, counts_ref, xbuf,
           xbf, oh_bf, oh_f32, sems):
    j = pl.program_id(0)
    p = pl.program_id(1)
    nb = pl.program_id(2)
    J = pl.num_programs(0)
    NCH = pl.num_programs(2)
    F = x_hbm.shape[1]
    C = x_hbm.shape[2]
    CH = F // NCH
    slot = jax.lax.rem(j, 2)

    def copy_in(jj):
        sl = jax.lax.rem(jj, 2)
        pltpu.make_async_copy(x_hbm.at[jj], xbuf.at[sl], sems.at[sl]).start()

    @pl.when(jnp.logical_and(p == 0, nb == 0))
    def _prefetch():
        @pl.when(j == 0)
        def _():
            copy_in(0)

        @pl.when(j + 1 < J)
        def _():
            copy_in(j + 1)

        pltpu.make_async_copy(x_hbm.at[j], xbuf.at[slot], sems.at[slot]).wait()

    @pl.when(jnp.logical_and(j == 0, p == 0))
    def _build_onehot():
        bi = bi_ref[0, :, pl.ds(nb * CH, CH)]
        seg_iota = jax.lax.broadcasted_iota(jnp.int32, (S, CH), 0)
        ohf = (seg_iota == bi).astype(jnp.float32)
        oh_f32[:, pl.ds(nb * CH, CH)] = ohf
        oh_bf[:, pl.ds(nb * CH, CH)] = ohf.astype(jnp.bfloat16)

        @pl.when(nb == 0)
        def _():
            counts_ref[...] = jax.lax.dot_general(
                cp_ref[...], jnp.ones((_NC, C), jnp.float32),
                (((0,), (0,)), ((), ())), preferred_element_type=jnp.float32)

    oh_b = oh_bf[:, pl.ds(nb * CH, CH)]

    @pl.when(p == 0)
    def _pass_a():
        x2 = xbuf[slot, pl.ds(nb * CH, CH), :].astype(jnp.bfloat16)
        xbf[pl.ds(nb * CH, CH), :] = x2

        @pl.when(nb == 0)
        def _():
            gc_ref[...] = jnp.zeros((S, C), jnp.float32)

        gc_ref[...] += jnp.dot(oh_b, x2, preferred_element_type=jnp.float32)

        @pl.when(nb == NCH - 1)
        def _():
            mean = gc_ref[...] / jnp.clip(counts_ref[...], 1.0, None)
            gc_ref[...] = jnp.tanh(
                jnp.dot(mean, w_ref[...], preferred_element_type=jnp.float32))

    @pl.when(p == 1)
    def _pass_b():
        x2 = xbf[pl.ds(nb * CH, CH), :]
        scores_t = jax.lax.dot_general(
            gc_ref[...].astype(jnp.bfloat16), x2, (((1,), (1,)), ((), ())),
            preferred_element_type=jnp.float32)
        s_row = jnp.sum(scores_t * oh_f32[:, pl.ds(nb * CH, CH)],
                        axis=0, keepdims=True)
        weighted = oh_b * jax.nn.sigmoid(s_row).astype(jnp.bfloat16)

        @pl.when(nb == 0)
        def _():
            out_ref[0] = jnp.zeros((S, C), jnp.float32)

        out_ref[0] += jnp.dot(weighted, x2, preferred_element_type=jnp.float32)

        @pl.when(nb == NCH - 1)
        def _():
            out_ref[0] = out_ref[0] / jnp.clip(counts_ref[...], 1.0, None)


def kernel(x, batch_index, weight):
    J, F, C = x.shape
    NCH = 1
    bi_flat = batch_index.astype(jnp.int32)
    counts_partial = _sc_histogram(F)(bi_flat)
    bi = bi_flat.reshape(1, 1, F)
    return pl.pallas_call(
        _fused,
        grid=(J, 2, NCH),
        in_specs=[
            pl.BlockSpec((1, 1, F), lambda j, p, nb: (0, 0, 0)),
            pl.BlockSpec((_NC, S), lambda j, p, nb: (0, 0)),
            pl.BlockSpec(memory_space=pl.ANY),
            pl.BlockSpec((C, C), lambda j, p, nb: (0, 0)),
        ],
        out_specs=pl.BlockSpec((1, S, C), lambda j, p, nb: (j, 0, 0)),
        out_shape=jax.ShapeDtypeStruct((J, S, C), jnp.float32),
        scratch_shapes=[
            pltpu.VMEM((S, C), jnp.float32),
            pltpu.VMEM((S, C), jnp.float32),
            pltpu.VMEM((2, F, C), jnp.float32),
            pltpu.VMEM((F, C), jnp.bfloat16),
            pltpu.VMEM((S, F), jnp.bfloat16),
            pltpu.VMEM((S, F), jnp.float32),
            pltpu.SemaphoreType.DMA((2,)),
        ],
    )(bi, counts_partial, x, weight)


# speedup vs baseline: 1.1572x; 1.1572x over previous
import jax
import jax.numpy as jnp
from jax.experimental import pallas as pl
from jax.experimental.pallas import tpu as pltpu

S = 16


def _fused(bi_ref, x_hbm, w_ref, out_ref, gc_ref, counts_ref, xbuf,
           xbf, oh_bf, oh_f32, sems):
    j = pl.program_id(0)
    p = pl.program_id(1)
    nb = pl.program_id(2)
    J = pl.num_programs(0)
    NCH = pl.num_programs(2)
    F = x_hbm.shape[1]
    C = x_hbm.shape[2]
    CH = F // NCH
    slot = jax.lax.rem(j, 2)

    def copy_in(jj):
        sl = jax.lax.rem(jj, 2)
        pltpu.make_async_copy(x_hbm.at[jj], xbuf.at[sl], sems.at[sl]).start()

    @pl.when(jnp.logical_and(p == 0, nb == 0))
    def _prefetch():
        @pl.when(j == 0)
        def _():
            copy_in(0)

        @pl.when(j + 1 < J)
        def _():
            copy_in(j + 1)

        pltpu.make_async_copy(x_hbm.at[j], xbuf.at[slot], sems.at[slot]).wait()

    @pl.when(jnp.logical_and(j == 0, p == 0))
    def _build_onehot():
        bi = bi_ref[0, :, pl.ds(nb * CH, CH)]
        seg_iota = jax.lax.broadcasted_iota(jnp.int32, (S, CH), 0)
        ohf = (seg_iota == bi).astype(jnp.float32)
        oh_f32[:, pl.ds(nb * CH, CH)] = ohf
        oh_bf[:, pl.ds(nb * CH, CH)] = ohf.astype(jnp.bfloat16)
        cnt = jnp.broadcast_to(jnp.sum(ohf, axis=1, keepdims=True), (S, C))

        @pl.when(nb == 0)
        def _():
            counts_ref[...] = jnp.zeros((S, C), jnp.float32)

        counts_ref[...] += cnt

    oh_b = oh_bf[:, pl.ds(nb * CH, CH)]

    @pl.when(p == 0)
    def _pass_a():
        x2 = xbuf[slot, pl.ds(nb * CH, CH), :].astype(jnp.bfloat16)
        xbf[pl.ds(nb * CH, CH), :] = x2

        @pl.when(nb == 0)
        def _():
            gc_ref[...] = jnp.zeros((S, C), jnp.float32)

        gc_ref[...] += jnp.dot(oh_b, x2, preferred_element_type=jnp.float32)

        @pl.when(nb == NCH - 1)
        def _():
            mean = gc_ref[...] / jnp.clip(counts_ref[...], 1.0, None)
            gc_ref[...] = jnp.tanh(
                jnp.dot(mean, w_ref[...], preferred_element_type=jnp.float32))

    @pl.when(p == 1)
    def _pass_b():
        x2 = xbf[pl.ds(nb * CH, CH), :]
        scores_t = jax.lax.dot_general(
            gc_ref[...].astype(jnp.bfloat16), x2, (((1,), (1,)), ((), ())),
            preferred_element_type=jnp.float32)
        s_row = jnp.sum(scores_t * oh_f32[:, pl.ds(nb * CH, CH)],
                        axis=0, keepdims=True)
        weighted = oh_b * jax.nn.sigmoid(s_row).astype(jnp.bfloat16)

        @pl.when(nb == 0)
        def _():
            out_ref[0] = jnp.zeros((S, C), jnp.float32)

        out_ref[0] += jnp.dot(weighted, x2, preferred_element_type=jnp.float32)

        @pl.when(nb == NCH - 1)
        def _():
            out_ref[0] = out_ref[0] / jnp.clip(counts_ref[...], 1.0, None)


def kernel(x, batch_index, weight):
    J, F, C = x.shape
    NCH = 1
    bi = batch_index.astype(jnp.int32).reshape(1, 1, F)
    return pl.pallas_call(
        _fused,
        grid=(J, 2, NCH),
        in_specs=[
            pl.BlockSpec((1, 1, F), lambda j, p, nb: (0, 0, 0)),
            pl.BlockSpec(memory_space=pl.ANY),
            pl.BlockSpec((C, C), lambda j, p, nb: (0, 0)),
        ],
        out_specs=pl.BlockSpec((1, S, C), lambda j, p, nb: (j, 0, 0)),
        out_shape=jax.ShapeDtypeStruct((J, S, C), jnp.float32),
        scratch_shapes=[
            pltpu.VMEM((S, C), jnp.float32),
            pltpu.VMEM((S, C), jnp.float32),
            pltpu.VMEM((2, F, C), jnp.float32),
            pltpu.VMEM((F, C), jnp.bfloat16),
            pltpu.VMEM((S, F), jnp.bfloat16),
            pltpu.VMEM((S, F), jnp.float32),
            pltpu.SemaphoreType.DMA((2,)),
        ],
    )(bi, x, weight)
